# baseline (speedup 1.0000x reference)
import functools

import jax
import jax.numpy as jnp
from jax import lax
from jax.experimental import pallas as pl
from jax.experimental.pallas import tpu as pltpu
from jax.experimental.pallas import tpu_sc as plsc

DIM = 64


def _rel_pos_sc(pe_flat, seq_len, dim):
    info = plsc.get_sparse_core_info()
    num_cores, num_subcores = info.num_cores, info.num_subcores
    num_workers = num_cores * num_subcores
    rows_per_worker = seq_len // num_workers
    row_elems = seq_len * dim

    n_phases = 4
    cols = seq_len // n_phases
    win_rows = cols + rows_per_worker

    mesh = plsc.VectorSubcoreMesh(core_axis_name="c", subcore_axis_name="s")

    fire = 16

    @functools.partial(
        pl.kernel,
        mesh=mesh,
        out_type=jax.ShapeDtypeStruct((seq_len * seq_len * dim,), jnp.float32),
        scratch_types=[
            pltpu.VMEM((win_rows * dim,), jnp.float32),
            pltpu.SemaphoreType.DMA,
        ],
    )
    def k(pe_hbm, out_hbm, win, sem):
        c = lax.axis_index("c")
        s = lax.axis_index("s")
        base = (c * num_subcores + s) * rows_per_worker
        top = base + rows_per_worker - 1

        for h in range(n_phases):
            w0 = (seq_len - 1) - top + h * cols
            pltpu.sync_copy(pe_hbm.at[pl.ds(w0 * dim, win_rows * dim)], win)

            def copy_desc(i):
                off = top - i
                return pltpu.make_async_copy(
                    win.at[pl.ds(off * dim, cols * dim)],
                    out_hbm.at[pl.ds(i * row_elems + h * cols * dim, cols * dim)],
                    sem,
                )

            def chunk(ci, carry):
                row0 = base + ci * fire
                for b in range(fire):
                    copy_desc(row0 + b).start()
                for b in range(fire):
                    copy_desc(row0 + b).wait()
                return carry

            lax.fori_loop(0, rows_per_worker // fire, chunk, 0)

    return k(pe_flat).reshape(seq_len, seq_len, dim)


def kernel(x, pe):
    seq_len = x.shape[2]
    pe_padded = jnp.pad(pe, ((0, 1), (0, 0)))
    return _rel_pos_sc(pe_padded.reshape(-1), seq_len, DIM)


# speedup vs baseline: 1.2185x; 1.2185x over previous
import functools

import jax
import jax.numpy as jnp
from jax import lax
from jax.experimental import pallas as pl
from jax.experimental.pallas import tpu as pltpu
from jax.experimental.pallas import tpu_sc as plsc

DIM = 64
SC_ROWS = 512
TC_BLOCK_ROWS = 8


def _sc_part(pe_padded, seq_len, dim):
    table_rows = pe_padded.shape[0]
    info = plsc.get_sparse_core_info()
    num_cores, num_subcores = info.num_cores, info.num_subcores
    num_workers = num_cores * num_subcores
    rows_per_worker = SC_ROWS // num_workers
    fire = min(16, rows_per_worker)

    mesh = plsc.VectorSubcoreMesh(core_axis_name="c", subcore_axis_name="s")

    @functools.partial(
        pl.kernel,
        mesh=mesh,
        out_type=jax.ShapeDtypeStruct((seq_len, seq_len, dim), jnp.float32),
        scratch_types=[
            pltpu.VMEM_SHARED((table_rows, dim), jnp.float32),
            pltpu.SemaphoreType.DMA,
        ],
    )
    def k(pe_hbm, out_hbm, pe_sh, sem):
        c = lax.axis_index("c")
        s = lax.axis_index("s")

        @pl.when(s == 0)
        def _():
            pltpu.sync_copy(pe_hbm, pe_sh)

        plsc.subcore_barrier()

        base = (c * num_subcores + s) * rows_per_worker

        def copy_desc(i):
            start = (seq_len - 1) - i
            return pltpu.make_async_copy(
                pe_sh.at[pl.ds(start, seq_len), :], out_hbm.at[i], sem
            )

        def chunk(ci, carry):
            row0 = base + ci * fire
            for b in range(fire):
                copy_desc(row0 + b).start()
            for b in range(fire):
                copy_desc(row0 + b).wait()
            return carry

        lax.fori_loop(0, rows_per_worker // fire, chunk, 0)

    return k(pe_padded)


def _tc_part(partial, pe_padded, seq_len, dim):
    tc_rows = seq_len - SC_ROWS
    rb = TC_BLOCK_ROWS

    def body(partial_ref, pe_ref, out_ref):
        pid = pl.program_id(0)
        for r in range(rb):
            start = (seq_len - 1) - (SC_ROWS + pid * rb + r)
            out_ref[r] = pe_ref[pl.ds(start, seq_len), :]

    return pl.pallas_call(
        body,
        grid=(tc_rows // rb,),
        in_specs=[
            pl.BlockSpec(memory_space=pl.ANY),
            pl.BlockSpec((pe_padded.shape[0], dim), lambda i: (0, 0)),
        ],
        out_specs=pl.BlockSpec((rb, seq_len, dim), lambda i: (i + SC_ROWS // rb, 0, 0)),
        out_shape=jax.ShapeDtypeStruct((seq_len, seq_len, dim), jnp.float32),
        input_output_aliases={0: 0},
    )(partial, pe_padded)


def kernel(x, pe):
    seq_len = x.shape[2]
    pe_padded = jnp.pad(pe, ((0, 1), (0, 0)))
    partial = _sc_part(pe_padded, seq_len, DIM)
    return _tc_part(partial, pe_padded, seq_len, DIM)
